# baseline (speedup 1.0000x reference)
import functools

import jax
import jax.numpy as jnp
from jax import lax
from jax.experimental import pallas as pl
from jax.experimental.pallas import tpu as pltpu
from jax.experimental.pallas import tpu_sc as plsc

BATCH = 4096
HIST = 50
DIM = 64
VOC0 = 1000000
VOC1 = 100000
NUM_WORKERS = 32
ROWS_PER_WORKER = BATCH // NUM_WORKERS
R = 4
N_CHUNKS = ROWS_PER_WORKER // R
SLOTS = R * 2 * HIST
HALF = R * HIST
GATHER_BLK = 40
N_GB = HALF // GATHER_BLK
GROUPS = SLOTS // 16
ROW_F = 2 * HIST * DIM


def _build_sc_call():
    mesh = plsc.VectorSubcoreMesh(core_axis_name="c", subcore_axis_name="s")

    @functools.partial(
        pl.kernel,
        mesh=mesh,
        compiler_params=pltpu.CompilerParams(needs_layout_passes=False),
        out_type=jax.ShapeDtypeStruct((BATCH, ROW_F), jnp.float32),
        scratch_types=[
            pltpu.VMEM((SLOTS,), jnp.int32),
            pltpu.VMEM((SLOTS,), jnp.int32),
            pltpu.VMEM((SLOTS,), jnp.int32),
            pltpu.VMEM((SLOTS, 2 * DIM), jnp.float32),
            pltpu.VMEM((R * ROW_F,), jnp.float32),
            pltpu.VMEM((16,), jnp.int32),
            pltpu.VMEM((16,), jnp.int32),
            pltpu.SemaphoreType.DMA,
        ],
    )
    def sc_kernel(idx0, idx1, t0, t1, out, idxv, gidx, bases, g, obuf,
                  colc, dstc, sem):
        wid = lax.axis_index("s") * 2 + lax.axis_index("c")
        flat0 = wid * (ROWS_PER_WORKER * HIST)
        row0 = wid * ROWS_PER_WORKER

        lane = lax.iota(jnp.int32, 16)
        for grp in range(GROUPS):
            s = lane + grp * 16
            t = 1 - lax.shift_right_logical(s - HALF, 31)
            sm = s - t * HALF
            r = lax.shift_right_logical(sm * 41944, 21)
            j = sm - r * HIST
            bases[pl.ds(grp * 16, 16)] = r * ROW_F + t * (HIST * DIM) + j * DIM

        def chunk_body(c, carry):
            o0 = flat0 + c * HALF
            b0 = row0 + c * R
            pltpu.sync_copy(idx0.at[pl.ds(o0, HALF)], idxv.at[pl.ds(0, HALF)])
            pltpu.sync_copy(idx1.at[pl.ds(o0, HALF)], idxv.at[pl.ds(HALF, HALF)])
            for grp in range(GROUPS):
                v = idxv[pl.ds(grp * 16, 16)]
                gidx[pl.ds(grp * 16, 16)] = lax.shift_right_logical(v, 1)
            handles = []
            for q in range(N_GB):
                handles.append(pltpu.async_copy(
                    t0.at[gidx.at[pl.ds(q * GATHER_BLK, GATHER_BLK)]],
                    g.at[pl.ds(q * GATHER_BLK, GATHER_BLK)], sem))
            for q in range(N_GB):
                off = HALF + q * GATHER_BLK
                handles.append(pltpu.async_copy(
                    t1.at[gidx.at[pl.ds(off, GATHER_BLK)]],
                    g.at[pl.ds(off, GATHER_BLK)], sem))
            for h in handles:
                h.wait()
            ones = jnp.ones((16,), jnp.int32)
            for grp in range(GROUPS):
                rows16 = lane + grp * 16
                v = idxv[pl.ds(grp * 16, 16)]
                h64 = lax.shift_left(jnp.bitwise_and(v, ones), 6)
                colc[...] = h64
                dstc[...] = bases[pl.ds(grp * 16, 16)]

                def k_body(k, carry, rows16=rows16):
                    cols = colc[...]
                    dst = dstc[...]
                    vals = plsc.load_gather(g, [rows16, cols])
                    plsc.store_scatter(obuf, [dst], vals)
                    colc[...] = cols + ones
                    dstc[...] = dst + ones
                    return carry

                lax.fori_loop(0, DIM, k_body, 0)
            for r in range(R):
                pltpu.sync_copy(obuf.at[pl.ds(r * ROW_F, ROW_F)],
                                out.at[b0 + r])
            return carry

        lax.fori_loop(0, N_CHUNKS, chunk_body, 0)

    return sc_kernel


_sc_call = _build_sc_call()


def kernel(embed_0, embed_1, table_0, table_1):
    idx0 = embed_0.astype(jnp.int32).reshape(-1)
    idx1 = embed_1.astype(jnp.int32).reshape(-1)
    t0v = table_0.reshape(VOC0 // 2, 2 * DIM)
    t1v = table_1.reshape(VOC1 // 2, 2 * DIM)
    return _sc_call(idx0, idx1, t0v, t1v)


# speedup vs baseline: 1.1781x; 1.1781x over previous
import functools

import jax
import jax.numpy as jnp
from jax import lax
from jax.experimental import pallas as pl
from jax.experimental.pallas import tpu as pltpu
from jax.experimental.pallas import tpu_sc as plsc

BATCH = 4096
HIST = 50
DIM = 64
VOC0 = 1000000
VOC1 = 100000
NUM_WORKERS = 32
ROWS_PER_WORKER = BATCH // NUM_WORKERS
R = 4
N_CHUNKS = ROWS_PER_WORKER // R
HALF_REAL = R * HIST
HALF_PAD = 208
SLOTS = 2 * HALF_PAD
GROUPS = SLOTS // 16
GB = 104
ROW_F = 2 * HIST * DIM
OBUF_F = R * ROW_F + 1024
DUMP = R * ROW_F


def _build_sc_call():
    mesh = plsc.VectorSubcoreMesh(core_axis_name="c", subcore_axis_name="s")

    @functools.partial(
        pl.kernel,
        mesh=mesh,
        compiler_params=pltpu.CompilerParams(needs_layout_passes=False),
        out_type=jax.ShapeDtypeStruct((BATCH, ROW_F), jnp.float32),
        scratch_types=[
            pltpu.VMEM((SLOTS,), jnp.int32),
            pltpu.VMEM((SLOTS,), jnp.int32),
            pltpu.VMEM((SLOTS,), jnp.int32),
            pltpu.VMEM((SLOTS,), jnp.int32),
            pltpu.VMEM((SLOTS,), jnp.int32),
            pltpu.VMEM((SLOTS,), jnp.int32),
            pltpu.VMEM((SLOTS, 2 * DIM), jnp.float32),
            pltpu.VMEM((OBUF_F,), jnp.float32),
            pltpu.VMEM((OBUF_F,), jnp.float32),
            pltpu.SemaphoreType.DMA,
            pltpu.SemaphoreType.DMA,
            pltpu.SemaphoreType.DMA,
        ],
    )
    def sc_kernel(idx0, idx1, t0, t1, out, idxv, gidx, h64a, h64b, rowids,
                  bases, g, obufa, obufb, sem_g, sem_oa, sem_ob):
        wid = lax.axis_index("s") * 2 + lax.axis_index("c")
        flat0 = wid * (ROWS_PER_WORKER * HIST)
        row0 = wid * ROWS_PER_WORKER
        h64 = (h64a, h64b)
        obuf = (obufa, obufb)
        sem_o = (sem_oa, sem_ob)

        lane = lax.iota(jnp.int32, 16)
        zeros = jnp.zeros((16,), jnp.int32)
        idxv[pl.ds(HALF_REAL, 16)] = zeros
        idxv[pl.ds(SLOTS - 16, 16)] = zeros
        for grp in range(GROUPS):
            s = lane + grp * 16
            rowids[pl.ds(grp * 16, 16)] = s
            t = 1 - lax.shift_right_logical(s - HALF_PAD, 31)
            u = s - t * HALF_PAD
            pf = 1 - lax.shift_right_logical(u - HALF_REAL, 31)
            r = lax.shift_right_logical(u * 41944, 21)
            j = u - r * HIST
            real = r * ROW_F + t * (HIST * DIM) + j * DIM
            pad = DUMP + (u - HALF_REAL) * DIM
            bases[pl.ds(grp * 16, 16)] = real * (1 - pf) + pad * pf

        def stage(c, p):
            o0 = flat0 + c * HALF_REAL
            pltpu.sync_copy(idx0.at[pl.ds(o0, HALF_REAL)],
                            idxv.at[pl.ds(0, HALF_REAL)])
            pltpu.sync_copy(idx1.at[pl.ds(o0, HALF_REAL)],
                            idxv.at[pl.ds(HALF_PAD, HALF_REAL)])
            ones = jnp.ones((16,), jnp.int32)
            for grp in range(GROUPS):
                v = idxv[pl.ds(grp * 16, 16)]
                gidx[pl.ds(grp * 16, 16)] = lax.shift_right_logical(v, 1)
                h64[p][pl.ds(grp * 16, 16)] = lax.shift_left(
                    jnp.bitwise_and(v, ones), 6)

        def fire_half(h):
            tab = t0 if h == 0 else t1
            base = h * HALF_PAD
            for q in range(2):
                off = base + q * GB
                pltpu.async_copy(
                    tab.at[gidx.at[pl.ds(off, GB)]],
                    g.at[pl.ds(off, GB)], sem_g)

        def recombine_half(h, p):
            def grp_body(gi, carry):
                rows16 = rowids[pl.ds(gi * 16, 16)]
                ho = h64[p][pl.ds(gi * 16, 16)]
                b16 = bases[pl.ds(gi * 16, 16)]
                for k in range(DIM):
                    vals = plsc.load_gather(g, [rows16, ho + k])
                    plsc.store_scatter(obuf[p], [b16 + k], vals)
                return carry

            lax.fori_loop(h * (GROUPS // 2), (h + 1) * (GROUPS // 2),
                          grp_body, 0)

        def drain_gathers():
            pltpu.make_async_copy(t0.at[pl.ds(0, SLOTS)], g, sem_g).wait()

        def drain_writeback(p):
            for r in range(R):
                pltpu.make_async_copy(
                    out.at[row0], obuf[p].at[pl.ds(r * ROW_F, ROW_F)],
                    sem_o[p]).wait()

        stage(0, 0)
        fire_half(0)
        fire_half(1)

        def body(i, carry):
            for p in (0, 1):
                c = i * 2 + p
                drain_gathers()

                @pl.when(c >= 2)
                def _():
                    drain_writeback(p)

                @pl.when(c < N_CHUNKS - 1)
                def _():
                    stage(c + 1, 1 - p)

                recombine_half(0, p)

                @pl.when(c < N_CHUNKS - 1)
                def _():
                    fire_half(0)

                recombine_half(1, p)

                @pl.when(c < N_CHUNKS - 1)
                def _():
                    fire_half(1)

                b0 = row0 + c * R
                for r in range(R):
                    pltpu.async_copy(
                        obuf[p].at[pl.ds(r * ROW_F, ROW_F)],
                        out.at[b0 + r], sem_o[p])
            return carry

        lax.fori_loop(0, N_CHUNKS // 2, body, 0)
        drain_writeback(0)
        drain_writeback(1)

    return sc_kernel


_sc_call = _build_sc_call()


def kernel(embed_0, embed_1, table_0, table_1):
    idx0 = embed_0.astype(jnp.int32).reshape(-1)
    idx1 = embed_1.astype(jnp.int32).reshape(-1)
    t0v = table_0.reshape(VOC0 // 2, 2 * DIM)
    t1v = table_1.reshape(VOC1 // 2, 2 * DIM)
    return _sc_call(idx0, idx1, t0v, t1v)


# speedup vs baseline: 1.5212x; 1.2912x over previous
import functools

import jax
import jax.numpy as jnp
from jax import lax
from jax.experimental import pallas as pl
from jax.experimental.pallas import tpu as pltpu
from jax.experimental.pallas import tpu_sc as plsc

BATCH = 4096
HIST = 50
DIM = 64
VOC0 = 1000000
VOC1 = 100000
NUM_WORKERS = 32
ROWS_PER_WORKER = BATCH // NUM_WORKERS
R = 4
N_CHUNKS = ROWS_PER_WORKER // R
HALF_REAL = R * HIST
HALF_PAD = 208
SLOTS = 2 * HALF_PAD
GROUPS = SLOTS // 16
GB = 104
ROW_F = 2 * HIST * DIM
OBUF_F = R * ROW_F + 1024
DUMP = R * ROW_F


def _build_sc_call():
    mesh = plsc.VectorSubcoreMesh(core_axis_name="c", subcore_axis_name="s")

    @functools.partial(
        pl.kernel,
        mesh=mesh,
        compiler_params=pltpu.CompilerParams(needs_layout_passes=False),
        out_type=jax.ShapeDtypeStruct((BATCH, ROW_F), jnp.float32),
        scratch_types=[
            pltpu.VMEM((SLOTS,), jnp.int32),
            pltpu.VMEM((SLOTS,), jnp.int32),
            pltpu.VMEM((SLOTS,), jnp.int32),
            pltpu.VMEM((SLOTS,), jnp.int32),
            pltpu.VMEM((SLOTS,), jnp.int32),
            pltpu.VMEM((SLOTS,), jnp.int32),
            pltpu.VMEM((SLOTS, 2 * DIM), jnp.float32),
            pltpu.VMEM((OBUF_F,), jnp.float32),
            pltpu.VMEM((OBUF_F,), jnp.float32),
            pltpu.SemaphoreType.DMA,
            pltpu.SemaphoreType.DMA,
            pltpu.SemaphoreType.DMA,
        ],
    )
    def sc_kernel(idx0, idx1, t0, t1, out, idxv, gidx, h64a, h64b, rowids,
                  bases, g, obufa, obufb, sem_g, sem_oa, sem_ob):
        wid = lax.axis_index("s") * 2 + lax.axis_index("c")
        flat0 = wid * (ROWS_PER_WORKER * HIST)
        row0 = wid * ROWS_PER_WORKER
        h64 = (h64a, h64b)
        obuf = (obufa, obufb)
        sem_o = (sem_oa, sem_ob)

        lane = lax.iota(jnp.int32, 16)
        zeros = jnp.zeros((16,), jnp.int32)
        idxv[pl.ds(HALF_REAL, 16)] = zeros
        idxv[pl.ds(SLOTS - 16, 16)] = zeros
        for grp in range(GROUPS):
            s = lane + grp * 16
            rowids[pl.ds(grp * 16, 16)] = s
            t = 1 - lax.shift_right_logical(s - HALF_PAD, 31)
            u = s - t * HALF_PAD
            pf = 1 - lax.shift_right_logical(u - HALF_REAL, 31)
            r = lax.shift_right_logical(u * 41944, 21)
            j = u - r * HIST
            real = r * ROW_F + t * (HIST * DIM) + j * DIM
            pad = DUMP + (u - HALF_REAL) * DIM
            bases[pl.ds(grp * 16, 16)] = real * (1 - pf) + pad * pf

        def stage(c, p):
            o0 = flat0 + c * HALF_REAL
            pltpu.sync_copy(idx0.at[pl.ds(o0, HALF_REAL)],
                            idxv.at[pl.ds(0, HALF_REAL)])
            pltpu.sync_copy(idx1.at[pl.ds(o0, HALF_REAL)],
                            idxv.at[pl.ds(HALF_PAD, HALF_REAL)])
            ones = jnp.ones((16,), jnp.int32)
            for grp in range(GROUPS):
                v = idxv[pl.ds(grp * 16, 16)]
                gidx[pl.ds(grp * 16, 16)] = lax.shift_right_logical(v, 1)
                h64[p][pl.ds(grp * 16, 16)] = lax.shift_left(
                    jnp.bitwise_and(v, ones), 6)

        def fire_half(h):
            tab = t0 if h == 0 else t1
            base = h * HALF_PAD
            for q in range(2):
                off = base + q * GB
                pltpu.async_copy(
                    tab.at[gidx.at[pl.ds(off, GB)]],
                    g.at[pl.ds(off, GB)], sem_g)

        def recombine_half(h, p):
            @plsc.parallel_loop(h * (GROUPS // 2), (h + 1) * (GROUPS // 2),
                                unroll=2)
            def grp_body(gi):
                rows16 = rowids[pl.ds(gi * 16, 16)]
                ho = h64[p][pl.ds(gi * 16, 16)]
                b16 = bases[pl.ds(gi * 16, 16)]
                for kb in range(0, DIM, 8):
                    vals = [plsc.load_gather(g, [rows16, ho + (kb + u)])
                            for u in range(8)]
                    for u in range(8):
                        plsc.store_scatter(obuf[p], [b16 + (kb + u)], vals[u])

        def drain_gathers():
            pltpu.make_async_copy(t0.at[pl.ds(0, SLOTS)], g, sem_g).wait()

        def drain_writeback(p):
            for r in range(R):
                pltpu.make_async_copy(
                    out.at[row0], obuf[p].at[pl.ds(r * ROW_F, ROW_F)],
                    sem_o[p]).wait()

        stage(0, 0)
        fire_half(0)
        fire_half(1)

        def body(i, carry):
            for p in (0, 1):
                c = i * 2 + p
                drain_gathers()

                @pl.when(c >= 2)
                def _():
                    drain_writeback(p)

                @pl.when(c < N_CHUNKS - 1)
                def _():
                    stage(c + 1, 1 - p)

                recombine_half(0, p)

                @pl.when(c < N_CHUNKS - 1)
                def _():
                    fire_half(0)

                recombine_half(1, p)

                @pl.when(c < N_CHUNKS - 1)
                def _():
                    fire_half(1)

                b0 = row0 + c * R
                for r in range(R):
                    pltpu.async_copy(
                        obuf[p].at[pl.ds(r * ROW_F, ROW_F)],
                        out.at[b0 + r], sem_o[p])
            return carry

        lax.fori_loop(0, N_CHUNKS // 2, body, 0)
        drain_writeback(0)
        drain_writeback(1)

    return sc_kernel


_sc_call = _build_sc_call()


def kernel(embed_0, embed_1, table_0, table_1):
    idx0 = embed_0.astype(jnp.int32).reshape(-1)
    idx1 = embed_1.astype(jnp.int32).reshape(-1)
    t0v = table_0.reshape(VOC0 // 2, 2 * DIM)
    t1v = table_1.reshape(VOC1 // 2, 2 * DIM)
    return _sc_call(idx0, idx1, t0v, t1v)


# speedup vs baseline: 1.7222x; 1.1321x over previous
import functools

import jax
import jax.numpy as jnp
from jax import lax
from jax.experimental import pallas as pl
from jax.experimental.pallas import tpu as pltpu
from jax.experimental.pallas import tpu_sc as plsc

BATCH = 4096
HIST = 50
DIM = 64
VOC0 = 1000000
VOC1 = 100000
NUM_WORKERS = 32
ROWS_PER_WORKER = BATCH // NUM_WORKERS
R = 4
N_CHUNKS = ROWS_PER_WORKER // R
HALF_REAL = R * HIST
HALF_PAD = 208
SLOTS = 2 * HALF_PAD
GROUPS = SLOTS // 16
GB = 104
ROW_F = 2 * HIST * DIM
OBUF_F = R * ROW_F + 1024
DUMP = R * ROW_F


def _build_sc_call():
    mesh = plsc.VectorSubcoreMesh(core_axis_name="c", subcore_axis_name="s")

    @functools.partial(
        pl.kernel,
        mesh=mesh,
        compiler_params=pltpu.CompilerParams(needs_layout_passes=False),
        out_type=jax.ShapeDtypeStruct((BATCH, ROW_F), jnp.float32),
        scratch_types=[
            pltpu.VMEM((SLOTS,), jnp.int32),
            pltpu.VMEM((SLOTS,), jnp.int32),
            pltpu.VMEM((SLOTS,), jnp.int32),
            pltpu.VMEM((SLOTS,), jnp.int32),
            pltpu.VMEM((SLOTS,), jnp.int32),
            pltpu.VMEM((SLOTS,), jnp.int32),
            pltpu.VMEM((SLOTS, 2 * DIM), jnp.float32),
            pltpu.VMEM((OBUF_F,), jnp.float32),
            pltpu.VMEM((OBUF_F,), jnp.float32),
            pltpu.SemaphoreType.DMA,
            pltpu.SemaphoreType.DMA,
            pltpu.SemaphoreType.DMA,
        ],
    )
    def sc_kernel(idx0, idx1, t0, t1, out, idxv, gidx, h64a, h64b, rowids,
                  bases, g, obufa, obufb, sem_g, sem_oa, sem_ob):
        wid = lax.axis_index("s") * 2 + lax.axis_index("c")
        flat0 = wid * (ROWS_PER_WORKER * HIST)
        row0 = wid * ROWS_PER_WORKER
        h64 = (h64a, h64b)
        obuf = (obufa, obufb)
        sem_o = (sem_oa, sem_ob)

        lane = lax.iota(jnp.int32, 16)
        zeros = jnp.zeros((16,), jnp.int32)
        idxv[pl.ds(HALF_REAL, 16)] = zeros
        idxv[pl.ds(SLOTS - 16, 16)] = zeros
        for grp in range(GROUPS):
            s = lane + grp * 16
            rowids[pl.ds(grp * 16, 16)] = s
            t = 1 - lax.shift_right_logical(s - HALF_PAD, 31)
            u = s - t * HALF_PAD
            pf = 1 - lax.shift_right_logical(u - HALF_REAL, 31)
            r = lax.shift_right_logical(u * 41944, 21)
            j = u - r * HIST
            real = r * ROW_F + t * (HIST * DIM) + j * DIM
            pad = DUMP + (u - HALF_REAL) * DIM
            bases[pl.ds(grp * 16, 16)] = real * (1 - pf) + pad * pf

        def stage(c, p):
            o0 = flat0 + c * HALF_REAL
            pltpu.sync_copy(idx0.at[pl.ds(o0, HALF_REAL)],
                            idxv.at[pl.ds(0, HALF_REAL)])
            pltpu.sync_copy(idx1.at[pl.ds(o0, HALF_REAL)],
                            idxv.at[pl.ds(HALF_PAD, HALF_REAL)])
            ones = jnp.ones((16,), jnp.int32)
            for grp in range(GROUPS):
                v = idxv[pl.ds(grp * 16, 16)]
                gidx[pl.ds(grp * 16, 16)] = lax.shift_right_logical(v, 1)
                h64[p][pl.ds(grp * 16, 16)] = lax.shift_left(
                    jnp.bitwise_and(v, ones), 6)

        def fire_half(h):
            tab = t0 if h == 0 else t1
            base = h * HALF_PAD
            for q in range(2):
                off = base + q * GB
                pltpu.async_copy(
                    tab.at[gidx.at[pl.ds(off, GB)]],
                    g.at[pl.ds(off, GB)], sem_g)

        def recombine_half(h, p):
            @plsc.parallel_loop(h * (GROUPS // 2), (h + 1) * (GROUPS // 2),
                                unroll=2)
            def grp_body(gi):
                rows16 = rowids[pl.ds(gi * 16, 16)]
                ho = h64[p][pl.ds(gi * 16, 16)]
                b16 = bases[pl.ds(gi * 16, 16)]
                for kb in range(0, DIM, 8):
                    kvs = [jnp.bitwise_and(lane + (kb + u), 63)
                           for u in range(8)]
                    vals = [plsc.load_gather(g, [rows16, ho + kvs[u]])
                            for u in range(8)]
                    for u in range(8):
                        plsc.store_scatter(obuf[p], [b16 + kvs[u]], vals[u])

        def drain_gathers():
            pltpu.make_async_copy(t0.at[pl.ds(0, SLOTS)], g, sem_g).wait()

        def drain_writeback(p):
            for r in range(R):
                pltpu.make_async_copy(
                    out.at[row0], obuf[p].at[pl.ds(r * ROW_F, ROW_F)],
                    sem_o[p]).wait()

        stage(0, 0)
        fire_half(0)
        fire_half(1)

        def body(i, carry):
            for p in (0, 1):
                c = i * 2 + p
                drain_gathers()

                @pl.when(c >= 2)
                def _():
                    drain_writeback(p)

                @pl.when(c < N_CHUNKS - 1)
                def _():
                    stage(c + 1, 1 - p)

                recombine_half(0, p)

                @pl.when(c < N_CHUNKS - 1)
                def _():
                    fire_half(0)

                recombine_half(1, p)

                @pl.when(c < N_CHUNKS - 1)
                def _():
                    fire_half(1)

                b0 = row0 + c * R
                for r in range(R):
                    pltpu.async_copy(
                        obuf[p].at[pl.ds(r * ROW_F, ROW_F)],
                        out.at[b0 + r], sem_o[p])
            return carry

        lax.fori_loop(0, N_CHUNKS // 2, body, 0)
        drain_writeback(0)
        drain_writeback(1)

    return sc_kernel


_sc_call = _build_sc_call()


def kernel(embed_0, embed_1, table_0, table_1):
    idx0 = embed_0.astype(jnp.int32).reshape(-1)
    idx1 = embed_1.astype(jnp.int32).reshape(-1)
    t0v = table_0.reshape(VOC0 // 2, 2 * DIM)
    t1v = table_1.reshape(VOC1 // 2, 2 * DIM)
    return _sc_call(idx0, idx1, t0v, t1v)
